# Initial kernel scaffold; baseline (speedup 1.0000x reference)
import functools

import jax
import jax.numpy as jnp
from jax import lax
from jax.experimental import pallas as pl
from jax.experimental.pallas import tpu as pltpu
from jax.experimental.pallas import tpu_sc as plsc

NUM_GENES = 1000000
EMBED_DIM = 64
B_TOTAL = 16384 * 50
NW = 32
PER_W = B_TOTAL // NW
CHUNK = 512
N_CHUNKS = PER_W // CHUNK


def _gather_kernel(idx_hbm, table_hbm, out_hbm, idx_v, rows_v, sem):
    wid = lax.axis_index("s") * 2 + lax.axis_index("c")
    base = wid * PER_W

    def body(i, carry):
        off = base + i * CHUNK
        pltpu.sync_copy(idx_hbm.at[pl.ds(off, CHUNK)], idx_v)
        pltpu.async_copy(table_hbm.at[idx_v], rows_v, sem).wait()
        pltpu.sync_copy(rows_v, out_hbm.at[pl.ds(off, CHUNK)])
        return carry

    lax.fori_loop(0, N_CHUNKS, body, 0)


@jax.jit
def _embed(gene_idx, table):
    idx_flat = gene_idx.reshape(-1)
    mesh = plsc.VectorSubcoreMesh(core_axis_name="c", subcore_axis_name="s")
    out = pl.kernel(
        _gather_kernel,
        mesh=mesh,
        out_type=jax.ShapeDtypeStruct((B_TOTAL, EMBED_DIM), jnp.float32),
        scratch_types=[
            pltpu.VMEM((CHUNK,), jnp.int32),
            pltpu.VMEM((CHUNK, EMBED_DIM), jnp.float32),
            pltpu.SemaphoreType.DMA,
        ],
    )(idx_flat, table)
    return out.reshape(gene_idx.shape[0], gene_idx.shape[1], EMBED_DIM)


def kernel(gene_idx, table):
    return _embed(gene_idx, table)


# speedup vs baseline: 1.7990x; 1.7990x over previous
import functools

import jax
import jax.numpy as jnp
from jax import lax
from jax.experimental import pallas as pl
from jax.experimental.pallas import tpu as pltpu
from jax.experimental.pallas import tpu_sc as plsc

NUM_GENES = 1000000
EMBED_DIM = 64
B_TOTAL = 16384 * 50
NW = 32
PER_W = B_TOTAL // NW
CHUNK = 512
N_CHUNKS = PER_W // CHUNK


def _gather_kernel(idx_hbm, table_hbm, out_hbm, idx_v, rows_v, sem):
    wid = lax.axis_index("s") * 2 + lax.axis_index("c")
    base = wid * PER_W

    def body(i, carry):
        off = base + i * CHUNK
        pltpu.sync_copy(idx_hbm.at[pl.ds(off, CHUNK)], idx_v)
        pltpu.async_copy(table_hbm.at[idx_v], rows_v, sem).wait()
        pltpu.sync_copy(rows_v, out_hbm.at[pl.ds(off, CHUNK)])
        return carry

    lax.fori_loop(0, N_CHUNKS, body, 0)


@jax.jit
def _embed(gene_idx, table):
    idx_flat = gene_idx.reshape(-1)
    mesh = plsc.VectorSubcoreMesh(core_axis_name="c", subcore_axis_name="s")
    out = pl.kernel(
        _gather_kernel,
        mesh=mesh,
        out_type=jax.ShapeDtypeStruct((B_TOTAL, EMBED_DIM), jnp.float32),
        scratch_types=[
            pltpu.VMEM((CHUNK,), jnp.int32),
            pltpu.VMEM((CHUNK, EMBED_DIM), jnp.float32),
            pltpu.SemaphoreType.DMA,
        ],
        compiler_params=pltpu.CompilerParams(use_tc_tiling_on_sc=False),
    )(idx_flat, table)
    return out.reshape(gene_idx.shape[0], gene_idx.shape[1], EMBED_DIM)


def kernel(gene_idx, table):
    return _embed(gene_idx, table)


# speedup vs baseline: 1.8643x; 1.0364x over previous
import jax
import jax.numpy as jnp
from jax import lax
from jax.experimental import pallas as pl
from jax.experimental.pallas import tpu as pltpu
from jax.experimental.pallas import tpu_sc as plsc

NUM_GENES = 1000000
EMBED_DIM = 64
B_TOTAL = 16384 * 50
NW = 32
PER_W = B_TOTAL // NW
CHUNK = 512
N_CHUNKS = PER_W // CHUNK
NBUF = 2
MAIN_ITERS = (N_CHUNKS - NBUF) // NBUF


def _gather_kernel(idx_hbm, table_hbm, out_hbm, idx_v, rows0, rows1, gs0, gs1,
                   os0, os1):
    rows = [rows0, rows1]
    gs = [gs0, gs1]
    osm = [os0, os1]
    wid = lax.axis_index("s") * 2 + lax.axis_index("c")
    base = wid * PER_W

    pltpu.sync_copy(idx_hbm.at[wid], idx_v)

    def gather(c, b):
        return pltpu.async_copy(table_hbm.at[idx_v.at[c]], rows[b], gs[b])

    def gather_wait(c, b):
        pltpu.make_async_copy(table_hbm.at[idx_v.at[c]], rows[b], gs[b]).wait()

    def store(c, b):
        dst = out_hbm.at[pl.ds(base + c * CHUNK, CHUNK)]
        return pltpu.async_copy(rows[b], dst, osm[b])

    def store_wait(c, b):
        dst = out_hbm.at[pl.ds(base + c * CHUNK, CHUNK)]
        pltpu.make_async_copy(rows[b], dst, osm[b]).wait()

    for b in range(NBUF):
        gather(b, b)

    def body(j, carry):
        for b in range(NBUF):
            c = j * NBUF + b
            gather_wait(c, b)
            store(c, b)
        for b in range(NBUF):
            c = j * NBUF + b
            store_wait(c, b)
            gather(c + NBUF, b)
        return carry

    lax.fori_loop(0, MAIN_ITERS, body, 0)

    for b in range(NBUF):
        c = N_CHUNKS - NBUF + b
        gather_wait(c, b)
        store(c, b)
    for b in range(NBUF):
        c = N_CHUNKS - NBUF + b
        store_wait(c, b)


@jax.jit
def _embed(gene_idx, table):
    idx3 = gene_idx.reshape(NW, N_CHUNKS, CHUNK)
    mesh = plsc.VectorSubcoreMesh(core_axis_name="c", subcore_axis_name="s")
    out = pl.kernel(
        _gather_kernel,
        mesh=mesh,
        out_type=jax.ShapeDtypeStruct((B_TOTAL, EMBED_DIM), jnp.float32),
        scratch_types=[
            pltpu.VMEM((N_CHUNKS, CHUNK), jnp.int32),
            pltpu.VMEM((CHUNK, EMBED_DIM), jnp.float32),
            pltpu.VMEM((CHUNK, EMBED_DIM), jnp.float32),
            pltpu.SemaphoreType.DMA,
            pltpu.SemaphoreType.DMA,
            pltpu.SemaphoreType.DMA,
            pltpu.SemaphoreType.DMA,
        ],
        compiler_params=pltpu.CompilerParams(use_tc_tiling_on_sc=False),
    )(idx3, table)
    return out.reshape(gene_idx.shape[0], gene_idx.shape[1], EMBED_DIM)


def kernel(gene_idx, table):
    return _embed(gene_idx, table)
